# baseline (speedup 1.0000x reference)
import functools

import jax
import jax.numpy as jnp
from jax import lax
from jax.experimental import pallas as pl
from jax.experimental.pallas import tpu as pltpu
from jax.experimental.pallas import tpu_sc as plsc

H = 128
NC, NS = 2, 16
NW = NC * NS
CHUNK = 128

_sc_mesh = functools.partial(
    plsc.VectorSubcoreMesh,
    core_axis_name="c", subcore_axis_name="s", num_cores=NC, num_subcores=NS,
)


def _ln(h, g, b):
    mu = jnp.mean(h, axis=-1, keepdims=True)
    c = h - mu
    var = jnp.mean(c * c, axis=-1, keepdims=True)
    return c * lax.rsqrt(var + 1e-5) * g + b


def _dot(a, b):
    return jnp.dot(a, b, preferred_element_type=jnp.float32)


def _enc_body(x_ref, w1_ref, w2_ref, v_ref, o_ref):
    h = _dot(x_ref[...], w1_ref[...]) + v_ref[0:1, :]
    h = _ln(h, v_ref[1:2, :], v_ref[2:3, :])
    h = h * jax.nn.sigmoid(h)
    o_ref[...] = _dot(h, w2_ref[...]) + v_ref[3:4, :]


def _enc2_body(x_ref, w1_ref, w2_ref, v_ref, o_ref, ob_ref):
    h = _dot(x_ref[...], w1_ref[...]) + v_ref[0:1, :]
    h = _ln(h, v_ref[1:2, :], v_ref[2:3, :])
    h = h * jax.nn.sigmoid(h)
    o = _dot(h, w2_ref[...]) + v_ref[3:4, :]
    o_ref[...] = o
    ob_ref[...] = o.astype(jnp.bfloat16)


def _unpun(x_i32):
    lo = lax.bitcast_convert_type(x_i32 << 16, jnp.float32)
    hi = lax.bitcast_convert_type(x_i32 & jnp.int32(-65536), jnp.float32)
    return lo, hi


def _edge_body(hs_ref, hd_ref, he_ref, w1se_ref, w1so_ref, w1de_ref,
               w1do_ref, w1e_ref, w2_ref, v_ref, o_ref):
    he = he_ref[...]
    se, so = _unpun(hs_ref[...])
    de, do = _unpun(hd_ref[...])
    bf = jnp.bfloat16
    h = _dot(se.astype(bf), w1se_ref[...]) + _dot(so.astype(bf), w1so_ref[...])
    h = h + _dot(de.astype(bf), w1de_ref[...]) + _dot(do.astype(bf), w1do_ref[...])
    h = h + _dot(he.astype(bf), w1e_ref[...]) + v_ref[0:1, :]
    h = _ln(h, v_ref[1:2, :], v_ref[2:3, :])
    h = h * jax.nn.sigmoid(h)
    o = _dot(h.astype(bf), w2_ref[...]) + v_ref[3:4, :] + he
    o_ref[...] = _ln(o, v_ref[4:5, :], v_ref[5:6, :])


def _node_body(hn_ref, ag_ref, bg_ref, w1n_ref, w1a_ref, w2_ref, v_ref,
               o_ref, ob_ref):
    hn = hn_ref[...]
    agg = (ag_ref[0] + ag_ref[1]) + (bg_ref[0] + bg_ref[1])
    h = _dot(hn, w1n_ref[...]) + _dot(agg, w1a_ref[...]) + v_ref[0:1, :]
    h = _ln(h, v_ref[1:2, :], v_ref[2:3, :])
    h = h * jax.nn.sigmoid(h)
    o = _dot(h, w2_ref[...]) + v_ref[3:4, :] + hn
    o = _ln(o, v_ref[4:5, :], v_ref[5:6, :])
    o_ref[...] = o
    ob_ref[...] = o.astype(jnp.bfloat16)


def _dec_body(h_ref, w1_ref, w2_ref, wd_ref, v_ref, o_ref):
    x = h_ref[...]
    h = _dot(x, w1_ref[...]) + v_ref[0:1, :]
    h = _ln(h, v_ref[1:2, :], v_ref[2:3, :])
    h = h * jax.nn.sigmoid(h)
    d = _dot(h, w2_ref[...]) + v_ref[3:4, :] + x
    o_ref[...] = _dot(d, wd_ref[...]) + v_ref[4:5, :]


def _elem_body(ha_ref, hb_ref, w1_ref, w2_ref, wd_ref, v_ref, o_ref):
    x = 0.5 * (ha_ref[...] + hb_ref[...])
    h = _dot(x, w1_ref[...]) + v_ref[0:1, :]
    h = _ln(h, v_ref[1:2, :], v_ref[2:3, :])
    h = h * jax.nn.sigmoid(h)
    d = _dot(h, w2_ref[...]) + v_ref[3:4, :] + x
    o_ref[...] = _dot(d, wd_ref[...]) + v_ref[4:5, :]


def _full(shape):
    return pl.BlockSpec(shape, lambda i: (0,) * len(shape))


def _rows(bs, w=H):
    return pl.BlockSpec((bs, w), lambda i: (i, 0))


def _tc_call(body, n_rows, bs, row_args, const_args, out_w=H):
    grid = (n_rows // bs,)
    in_specs = [_rows(bs, a.shape[1]) for a in row_args]
    in_specs += [_full(a.shape) for a in const_args]
    return pl.pallas_call(
        body,
        grid=grid,
        in_specs=in_specs,
        out_specs=_rows(bs, out_w),
        out_shape=jax.ShapeDtypeStruct((n_rows, out_w), jnp.float32),
    )(*row_args, *const_args)


GB = 80
NQ = 4


def _sc_gather(table, src3, dst3):
    n_chunks = src3.shape[1]
    per_tile = n_chunks * GB
    ep = NW * per_tile
    dt, dw = table.dtype, table.shape[1]

    @functools.partial(
        pl.kernel,
        out_type=(jax.ShapeDtypeStruct((ep, dw), dt),
                  jax.ShapeDtypeStruct((ep, dw), dt)),
        mesh=_sc_mesh(),
        scratch_types=(
            [pltpu.VMEM((n_chunks, GB), jnp.int32)] * 2
            + [pltpu.VMEM((GB, dw), dt)] * (2 * NQ)
            + [pltpu.SemaphoreType.DMA] * 2
        ),
        compiler_params=pltpu.CompilerParams(use_tc_tiling_on_sc=False,
                                             skip_device_barrier=True),
    )
    def k(tab, s_idx, d_idx, o_s, o_d, iv_s, iv_d, *rest):
        bufs, (sem_g, sem_w) = rest[:2 * NQ], rest[2 * NQ:]
        wid = lax.axis_index("s") * NC + lax.axis_index("c")
        base = wid * per_tile
        pltpu.sync_copy(s_idx.at[wid], iv_s)
        pltpu.sync_copy(d_idx.at[wid], iv_d)

        @pl.loop(0, n_chunks, step=NQ)
        def _(ch):
            gets = []
            for q in range(NQ):
                gets.append(pltpu.async_copy(
                    tab.at[iv_s.at[ch + q]], bufs[2 * q], sem_g))
                gets.append(pltpu.async_copy(
                    tab.at[iv_d.at[ch + q]], bufs[2 * q + 1], sem_g))
            puts = []
            for q in range(NQ):
                gets[2 * q].wait()
                off = base + (ch + q) * GB
                puts.append(pltpu.async_copy(
                    bufs[2 * q], o_s.at[pl.ds(off, GB)], sem_w))
                gets[2 * q + 1].wait()
                puts.append(pltpu.async_copy(
                    bufs[2 * q + 1], o_d.at[pl.ds(off, GB)], sem_w))
            for p in puts:
                p.wait()

    return k(table, src3, dst3)


def _sc_scatter_add(h_edge, dst3, zeros_np):
    n_chunks = dst3.shape[1]
    per_tile = n_chunks * GB
    np_rows = zeros_np.shape[0]
    rows_per_sub = np_rows // NS

    @functools.partial(
        pl.kernel,
        out_type=jax.ShapeDtypeStruct((NC, np_rows, H), jnp.float32),
        mesh=_sc_mesh(),
        scratch_types=(
            [pltpu.VMEM_SHARED((np_rows, H), jnp.float32),
             pltpu.VMEM((n_chunks, GB), jnp.int32)]
            + [pltpu.VMEM((GB, H), jnp.float32)] * NQ
            + [pltpu.SemaphoreType.DMA] * 2
        ),
        compiler_params=pltpu.CompilerParams(skip_device_barrier=True),
    )
    def k(he, d_idx, zz, out, shared, iv, *rest):
        bufs, (sem_l, sem_s) = rest[:NQ], rest[NQ:]
        cid = lax.axis_index("c")
        sid = lax.axis_index("s")
        wid = sid * NC + cid
        rbase = sid * rows_per_sub
        zcp = pltpu.async_copy(zz.at[pl.ds(rbase, rows_per_sub)],
                               shared.at[pl.ds(rbase, rows_per_sub)], sem_l)
        pltpu.sync_copy(d_idx.at[wid], iv)
        zcp.wait()
        plsc.subcore_barrier()

        @pl.loop(0, n_chunks, step=NQ)
        def _(ch):
            loads = []
            for q in range(NQ):
                off = wid * per_tile + (ch + q) * GB
                loads.append(pltpu.async_copy(
                    he.at[pl.ds(off, GB)], bufs[q], sem_l))
            adds = []
            for q in range(NQ):
                loads[q].wait()
                adds.append(pltpu.async_copy(
                    bufs[q], shared.at[iv.at[ch + q]], sem_s, add=True))
            for a in adds:
                a.wait()

        plsc.subcore_barrier()
        pltpu.sync_copy(shared.at[pl.ds(rbase, rows_per_sub)],
                        out.at[cid, pl.ds(rbase, rows_per_sub)])

    return k(h_edge, dst3, zeros_np)


def _pad128(w, b):
    return jnp.zeros((H, H), jnp.float32).at[: w.shape[0]].set(w), b


def _vecs(*rows):
    v = jnp.zeros((8, H), jnp.float32)
    for i, r in enumerate(rows):
        v = v.at[i, : r.shape[0]].set(r)
    return v


def kernel(x, edge_attr, edge_index, params):
    n, e2 = x.shape[0], edge_attr.shape[0]
    npad = 10240
    eh = e2 // 2
    ehp = 81920
    n_chunks = ehp // NW // GB

    xp = jnp.zeros((npad, H), jnp.float32).at[:n, : x.shape[1]].set(x)
    ea_w = edge_attr.shape[1]
    eaps = [jnp.zeros((ehp, H), jnp.float32).at[:eh, :ea_w].set(
        edge_attr[i * eh:(i + 1) * eh]) for i in range(2)]
    srcs, dsts = [], []
    for i in range(2):
        s = jnp.zeros((ehp,), jnp.int32).at[:eh].set(
            edge_index[0, i * eh:(i + 1) * eh])
        d = jnp.full((ehp,), n, jnp.int32).at[:eh].set(
            edge_index[1, i * eh:(i + 1) * eh])
        srcs.append(s.reshape(NW, n_chunks, GB))
        dsts.append(d.reshape(NW, n_chunks, GB))
    zeros_np = jnp.zeros((npad, H), jnp.float32)

    pe = params["node_enc"]
    w1, _ = _pad128(pe["W1"], None)
    h_node, h_node_b = pl.pallas_call(
        _enc2_body,
        grid=(npad // 1024,),
        in_specs=[_rows(1024), _full((H, H)), _full((H, H)), _full((8, H))],
        out_specs=[_rows(1024), _rows(1024)],
        out_shape=[jax.ShapeDtypeStruct((npad, H), jnp.float32),
                   jax.ShapeDtypeStruct((npad, H), jnp.bfloat16)],
    )(xp, w1, pe["W2"], _vecs(pe["b1"], pe["g1"], pe["bt1"], pe["b2"]))
    pe = params["edge_enc"]
    w1, _ = _pad128(pe["W1"], None)
    evec = _vecs(pe["b1"], pe["g1"], pe["bt1"], pe["b2"])
    h_edges = [_tc_call(_enc_body, ehp, 2048, [eaps[i]], [w1, pe["W2"], evec])
               for i in range(2)]

    bf = jnp.bfloat16
    for blk in params["procs"]:
        pm = blk["edge_mlp"]
        w1s, w1d, w1e = pm["W1"][:H], pm["W1"][H:2 * H], pm["W1"][2 * H:]
        tab_i = lax.bitcast_convert_type(
            h_node_b.reshape(npad, H // 2, 2), jnp.int32)
        gath = [_sc_gather(tab_i, srcs[i], dsts[i]) for i in range(2)]
        ewts = [w1s[0::2].astype(bf), w1s[1::2].astype(bf),
                w1d[0::2].astype(bf), w1d[1::2].astype(bf),
                w1e.astype(bf), pm["W2"].astype(bf),
                _vecs(pm["b1"], pm["g1"], pm["bt1"], pm["b2"],
                      blk["eg"], blk["eb"])]
        h_edges = [_tc_call(_edge_body, ehp, 2048,
                            [gath[i][0], gath[i][1], h_edges[i]], ewts)
                   for i in range(2)]
        aggs = [_sc_scatter_add(h_edges[i], dsts[i], zeros_np)
                for i in range(2)]

        pm = blk["node_mlp"]
        w1n, w1a = pm["W1"][:H], pm["W1"][H:]
        agg_spec = pl.BlockSpec((NC, 1024, H), lambda i: (0, i, 0))
        h_node, h_node_b = pl.pallas_call(
            _node_body,
            grid=(npad // 1024,),
            in_specs=[_rows(1024), agg_spec, agg_spec,
                      _full((H, H)), _full((H, H)), _full((H, H)),
                      _full((8, H))],
            out_specs=[_rows(1024), _rows(1024)],
            out_shape=[jax.ShapeDtypeStruct((npad, H), jnp.float32),
                       jax.ShapeDtypeStruct((npad, H), jnp.bfloat16)],
        )(h_node, aggs[0], aggs[1], w1n, w1a, pm["W2"],
          _vecs(pm["b1"], pm["g1"], pm["bt1"], pm["b2"], blk["ng"], blk["nb"]))

    pd = params["node_dec_mlp"]
    wd = jnp.zeros((H, H), jnp.float32).at[:, :6].set(params["node_dec_W"])
    bd = jnp.zeros((H,), jnp.float32).at[:6].set(params["node_dec_b"])
    node_out = _tc_call(_dec_body, npad, 1024, [h_node],
                        [pd["W1"], pd["W2"], wd,
                         _vecs(pd["b1"], pd["g1"], pd["bt1"], pd["b2"], bd)])

    pd = params["elem_dec_mlp"]
    wd = jnp.zeros((H, H), jnp.float32).at[:, :7].set(params["elem_dec_W"])
    bd = jnp.zeros((H,), jnp.float32).at[:7].set(params["elem_dec_b"])
    elem_out = pl.pallas_call(
        _elem_body,
        grid=(eh // 1600,),
        in_specs=[_rows(1600), _rows(1600),
                  _full((H, H)), _full((H, H)), _full((H, H)), _full((8, H))],
        out_specs=_rows(1600),
        out_shape=jax.ShapeDtypeStruct((eh, H), jnp.float32),
    )(h_edges[0], h_edges[1], pd["W1"], pd["W2"], wd,
      _vecs(pd["b1"], pd["g1"], pd["bt1"], pd["b2"], bd))

    return (node_out[:n, :6], elem_out[:, :7])


# speedup vs baseline: 1.0008x; 1.0008x over previous
import functools

import jax
import jax.numpy as jnp
from jax import lax
from jax.experimental import pallas as pl
from jax.experimental.pallas import tpu as pltpu
from jax.experimental.pallas import tpu_sc as plsc

H = 128
NC, NS = 2, 16
NW = NC * NS

_sc_mesh = functools.partial(
    plsc.VectorSubcoreMesh,
    core_axis_name="c", subcore_axis_name="s", num_cores=NC, num_subcores=NS,
)


def _ln(h, g, b):
    mu = jnp.mean(h, axis=-1, keepdims=True)
    c = h - mu
    var = jnp.mean(c * c, axis=-1, keepdims=True)
    return c * lax.rsqrt(var + 1e-5) * g + b


def _dot(a, b):
    return jnp.dot(a, b, preferred_element_type=jnp.float32)


def _enc_body(x_ref, w1_ref, w2_ref, v_ref, o_ref):
    h = _dot(x_ref[...], w1_ref[...]) + v_ref[0:1, :]
    h = _ln(h, v_ref[1:2, :], v_ref[2:3, :])
    h = h * jax.nn.sigmoid(h)
    o_ref[...] = _dot(h, w2_ref[...]) + v_ref[3:4, :]


def _enc2_body(x_ref, w1_ref, w2_ref, v_ref, o_ref, ob_ref):
    h = _dot(x_ref[...], w1_ref[...]) + v_ref[0:1, :]
    h = _ln(h, v_ref[1:2, :], v_ref[2:3, :])
    h = h * jax.nn.sigmoid(h)
    o = _dot(h, w2_ref[...]) + v_ref[3:4, :]
    o_ref[...] = o
    ob_ref[...] = o.astype(jnp.bfloat16)


def _unpun(x_i32):
    lo = lax.bitcast_convert_type(x_i32 << 16, jnp.float32)
    hi = lax.bitcast_convert_type(x_i32 & jnp.int32(-65536), jnp.float32)
    return lo, hi


def _edge_body(hs_ref, hd_ref, he_ref, w1se_ref, w1so_ref, w1de_ref,
               w1do_ref, w1e_ref, w2_ref, v_ref, o_ref):
    he = he_ref[...]
    se, so = _unpun(hs_ref[...])
    de, do = _unpun(hd_ref[...])
    bf = jnp.bfloat16
    h = _dot(se.astype(bf), w1se_ref[...]) + _dot(so.astype(bf), w1so_ref[...])
    h = h + _dot(de.astype(bf), w1de_ref[...]) + _dot(do.astype(bf), w1do_ref[...])
    h = h + _dot(he.astype(bf), w1e_ref[...]) + v_ref[0:1, :]
    h = _ln(h, v_ref[1:2, :], v_ref[2:3, :])
    h = h * jax.nn.sigmoid(h)
    o = _dot(h.astype(bf), w2_ref[...]) + v_ref[3:4, :] + he
    o_ref[...] = _ln(o, v_ref[4:5, :], v_ref[5:6, :])


def _node_body(hn_ref, ag_ref, bg_ref, w1n_ref, w1a_ref, w2_ref, v_ref,
               o_ref, ob_ref):
    hn = hn_ref[...]
    agg = (ag_ref[0] + ag_ref[1]) + (bg_ref[0] + bg_ref[1])
    h = _dot(hn, w1n_ref[...]) + _dot(agg, w1a_ref[...]) + v_ref[0:1, :]
    h = _ln(h, v_ref[1:2, :], v_ref[2:3, :])
    h = h * jax.nn.sigmoid(h)
    o = _dot(h, w2_ref[...]) + v_ref[3:4, :] + hn
    o = _ln(o, v_ref[4:5, :], v_ref[5:6, :])
    o_ref[...] = o
    ob_ref[...] = o.astype(jnp.bfloat16)


def _dec_body(h_ref, w1_ref, w2_ref, wd_ref, v_ref, o_ref):
    x = h_ref[...]
    h = _dot(x, w1_ref[...]) + v_ref[0:1, :]
    h = _ln(h, v_ref[1:2, :], v_ref[2:3, :])
    h = h * jax.nn.sigmoid(h)
    d = _dot(h, w2_ref[...]) + v_ref[3:4, :] + x
    o_ref[...] = _dot(d, wd_ref[...]) + v_ref[4:5, :]


def _elem_body(ha_ref, hb_ref, w1_ref, w2_ref, wd_ref, v_ref, o_ref):
    x = 0.5 * (ha_ref[...] + hb_ref[...])
    h = _dot(x, w1_ref[...]) + v_ref[0:1, :]
    h = _ln(h, v_ref[1:2, :], v_ref[2:3, :])
    h = h * jax.nn.sigmoid(h)
    d = _dot(h, w2_ref[...]) + v_ref[3:4, :] + x
    o_ref[...] = _dot(d, wd_ref[...]) + v_ref[4:5, :]


def _full(shape):
    return pl.BlockSpec(shape, lambda i: (0,) * len(shape))


def _rows(bs, w=H):
    return pl.BlockSpec((bs, w), lambda i: (i, 0))


def _tc_call(body, n_rows, bs, row_args, const_args, out_w=H):
    grid = (n_rows // bs,)
    in_specs = [_rows(bs, a.shape[1]) for a in row_args]
    in_specs += [_full(a.shape) for a in const_args]
    return pl.pallas_call(
        body,
        grid=grid,
        in_specs=in_specs,
        out_specs=_rows(bs, out_w),
        out_shape=jax.ShapeDtypeStruct((n_rows, out_w), jnp.float32),
    )(*row_args, *const_args)


GB = 80
NQ = 4


def _sc_gather(table, src3, dst3):
    n_chunks = src3.shape[1]
    per_tile = n_chunks * GB
    ep = NW * per_tile
    dt, dw = table.dtype, table.shape[1]

    @functools.partial(
        pl.kernel,
        out_type=(jax.ShapeDtypeStruct((ep, dw), dt),
                  jax.ShapeDtypeStruct((ep, dw), dt)),
        mesh=_sc_mesh(),
        scratch_types=(
            [pltpu.VMEM((n_chunks, GB), jnp.int32)] * 2
            + [pltpu.VMEM((GB, dw), dt)] * (2 * NQ)
            + [pltpu.SemaphoreType.DMA] * 2
        ),
        compiler_params=pltpu.CompilerParams(use_tc_tiling_on_sc=False,
                                             skip_device_barrier=True),
    )
    def k(tab, s_idx, d_idx, o_s, o_d, iv_s, iv_d, *rest):
        bufs, (sem_g, sem_w) = rest[:2 * NQ], rest[2 * NQ:]
        wid = lax.axis_index("s") * NC + lax.axis_index("c")
        base = wid * per_tile
        pltpu.sync_copy(s_idx.at[wid], iv_s)
        pltpu.sync_copy(d_idx.at[wid], iv_d)

        @pl.loop(0, n_chunks, step=NQ)
        def _(ch):
            gets = []
            for q in range(NQ):
                gets.append(pltpu.async_copy(
                    tab.at[iv_s.at[ch + q]], bufs[2 * q], sem_g))
                gets.append(pltpu.async_copy(
                    tab.at[iv_d.at[ch + q]], bufs[2 * q + 1], sem_g))
            puts = []
            for q in range(NQ):
                gets[2 * q].wait()
                off = base + (ch + q) * GB
                puts.append(pltpu.async_copy(
                    bufs[2 * q], o_s.at[pl.ds(off, GB)], sem_w))
                gets[2 * q + 1].wait()
                puts.append(pltpu.async_copy(
                    bufs[2 * q + 1], o_d.at[pl.ds(off, GB)], sem_w))
            for p in puts:
                p.wait()

    return k(table, src3, dst3)


def _sc_scatter_add(h_edge, dst3, zeros_np):
    n_chunks = dst3.shape[1]
    per_tile = n_chunks * GB
    np_rows = zeros_np.shape[0]
    rows_per_sub = np_rows // NS

    @functools.partial(
        pl.kernel,
        out_type=jax.ShapeDtypeStruct((NC, np_rows, H), jnp.float32),
        mesh=_sc_mesh(),
        scratch_types=(
            [pltpu.VMEM_SHARED((np_rows, H), jnp.float32),
             pltpu.VMEM((n_chunks, GB), jnp.int32)]
            + [pltpu.VMEM((GB, H), jnp.float32)] * NQ
            + [pltpu.SemaphoreType.DMA] * 2
        ),
        compiler_params=pltpu.CompilerParams(skip_device_barrier=True),
    )
    def k(he, d_idx, zz, out, shared, iv, *rest):
        bufs, (sem_l, sem_s) = rest[:NQ], rest[NQ:]
        cid = lax.axis_index("c")
        sid = lax.axis_index("s")
        wid = sid * NC + cid
        rbase = sid * rows_per_sub
        zcp = pltpu.async_copy(zz.at[pl.ds(rbase, rows_per_sub)],
                               shared.at[pl.ds(rbase, rows_per_sub)], sem_l)
        pltpu.sync_copy(d_idx.at[wid], iv)
        zcp.wait()
        plsc.subcore_barrier()

        @pl.loop(0, n_chunks, step=NQ)
        def _(ch):
            loads = []
            for q in range(NQ):
                off = wid * per_tile + (ch + q) * GB
                loads.append(pltpu.async_copy(
                    he.at[pl.ds(off, GB)], bufs[q], sem_l))
            adds = []
            for q in range(NQ):
                loads[q].wait()
                adds.append(pltpu.async_copy(
                    bufs[q], shared.at[iv.at[ch + q]], sem_s, add=True))
            for a in adds:
                a.wait()

        plsc.subcore_barrier()
        pltpu.sync_copy(shared.at[pl.ds(rbase, rows_per_sub)],
                        out.at[cid, pl.ds(rbase, rows_per_sub)])

    return k(h_edge, dst3, zeros_np)


def _pad128(w, b):
    return jnp.zeros((H, H), jnp.float32).at[: w.shape[0]].set(w), b


def _vecs(*rows):
    v = jnp.zeros((8, H), jnp.float32)
    for i, r in enumerate(rows):
        v = v.at[i, : r.shape[0]].set(r)
    return v


def kernel(x, edge_attr, edge_index, params):
    n, e2 = x.shape[0], edge_attr.shape[0]
    npad = 10240
    eh = e2 // 2
    ehp = 81920
    n_chunks = ehp // NW // GB

    xp = jnp.zeros((npad, H), jnp.float32).at[:n, : x.shape[1]].set(x)
    ea_w = edge_attr.shape[1]
    eaps = [jnp.zeros((ehp, H), jnp.float32).at[:eh, :ea_w].set(
        edge_attr[i * eh:(i + 1) * eh]) for i in range(2)]
    srcs, dsts = [], []
    for i in range(2):
        s = jnp.zeros((ehp,), jnp.int32).at[:eh].set(
            edge_index[0, i * eh:(i + 1) * eh])
        d = jnp.full((ehp,), n, jnp.int32).at[:eh].set(
            edge_index[1, i * eh:(i + 1) * eh])
        srcs.append(s.reshape(NW, n_chunks, GB))
        dsts.append(d.reshape(NW, n_chunks, GB))
    zeros_np = jnp.zeros((npad, H), jnp.float32)

    pe = params["node_enc"]
    w1, _ = _pad128(pe["W1"], None)
    h_node, h_node_b = pl.pallas_call(
        _enc2_body,
        grid=(npad // 1024,),
        in_specs=[_rows(1024), _full((H, H)), _full((H, H)), _full((8, H))],
        out_specs=[_rows(1024), _rows(1024)],
        out_shape=[jax.ShapeDtypeStruct((npad, H), jnp.float32),
                   jax.ShapeDtypeStruct((npad, H), jnp.bfloat16)],
    )(xp, w1, pe["W2"], _vecs(pe["b1"], pe["g1"], pe["bt1"], pe["b2"]))
    pe = params["edge_enc"]
    w1, _ = _pad128(pe["W1"], None)
    evec = _vecs(pe["b1"], pe["g1"], pe["bt1"], pe["b2"])
    h_edges = [_tc_call(_enc_body, ehp, 2048, [eaps[i]], [w1, pe["W2"], evec])
               for i in range(2)]

    bf = jnp.bfloat16
    for blk in params["procs"]:
        pm = blk["edge_mlp"]
        w1s, w1d, w1e = pm["W1"][:H], pm["W1"][H:2 * H], pm["W1"][2 * H:]
        tab_i = lax.bitcast_convert_type(
            h_node_b.reshape(npad, H // 2, 2), jnp.int32)
        gath = [_sc_gather(tab_i, srcs[i], dsts[i]) for i in range(2)]
        ewts = [w1s[0::2].astype(bf), w1s[1::2].astype(bf),
                w1d[0::2].astype(bf), w1d[1::2].astype(bf),
                w1e.astype(bf), pm["W2"].astype(bf),
                _vecs(pm["b1"], pm["g1"], pm["bt1"], pm["b2"],
                      blk["eg"], blk["eb"])]
        h_edges = [_tc_call(_edge_body, ehp, 2048,
                            [gath[i][0], gath[i][1], h_edges[i]], ewts)
                   for i in range(2)]
        aggs = [_sc_scatter_add(h_edges[i], dsts[i], zeros_np)
                for i in range(2)]

        pm = blk["node_mlp"]
        w1n, w1a = pm["W1"][:H], pm["W1"][H:]
        agg_spec = pl.BlockSpec((NC, 1024, H), lambda i: (0, i, 0))
        h_node, h_node_b = pl.pallas_call(
            _node_body,
            grid=(npad // 1024,),
            in_specs=[_rows(1024), agg_spec, agg_spec,
                      _full((H, H)), _full((H, H)), _full((H, H)),
                      _full((8, H))],
            out_specs=[_rows(1024), _rows(1024)],
            out_shape=[jax.ShapeDtypeStruct((npad, H), jnp.float32),
                       jax.ShapeDtypeStruct((npad, H), jnp.bfloat16)],
        )(h_node, aggs[0], aggs[1], w1n, w1a, pm["W2"],
          _vecs(pm["b1"], pm["g1"], pm["bt1"], pm["b2"], blk["ng"], blk["nb"]))

    pd = params["node_dec_mlp"]
    wd = jnp.zeros((H, H), jnp.float32).at[:, :6].set(params["node_dec_W"])
    bd = jnp.zeros((H,), jnp.float32).at[:6].set(params["node_dec_b"])
    node_out = _tc_call(_dec_body, npad, 1024, [h_node],
                        [pd["W1"], pd["W2"], wd,
                         _vecs(pd["b1"], pd["g1"], pd["bt1"], pd["b2"], bd)])

    pd = params["elem_dec_mlp"]
    wd = jnp.zeros((H, H), jnp.float32).at[:, :7].set(params["elem_dec_W"])
    bd = jnp.zeros((H,), jnp.float32).at[:7].set(params["elem_dec_b"])
    elem_out = pl.pallas_call(
        _elem_body,
        grid=(eh // 1600,),
        in_specs=[_rows(1600), _rows(1600),
                  _full((H, H)), _full((H, H)), _full((H, H)), _full((8, H))],
        out_specs=_rows(1600),
        out_shape=jax.ShapeDtypeStruct((eh, H), jnp.float32),
    )(h_edges[0], h_edges[1], pd["W1"], pd["W2"], wd,
      _vecs(pd["b1"], pd["g1"], pd["bt1"], pd["b2"], bd))

    return (node_out[:n, :6], elem_out[:, :7])
